# baseline (speedup 1.0000x reference)
import jax
import jax.numpy as jnp
from jax.experimental import pallas as pl
from jax.experimental.pallas import tpu as pltpu

NUM_CLASSES = 1000
B = 1024
S = 50


CBLK = 200


def _onehot_block(x_ref, o_ref):
    j = pl.program_id(1)
    c = jax.lax.broadcasted_iota(jnp.int32, (1, CBLK, B), 1) + j * CBLK
    o_ref[...] = (c == x_ref[...]).astype(jnp.int32)


def kernel(x):
    xt = x.T.reshape(S, 1, B)
    out_t = pl.pallas_call(
        _onehot_block,
        grid=(S, NUM_CLASSES // CBLK),
        in_specs=[pl.BlockSpec((1, 1, B), lambda s, j: (s, 0, 0))],
        out_specs=pl.BlockSpec((1, CBLK, B), lambda s, j: (s, j, 0)),
        out_shape=jax.ShapeDtypeStruct((S, NUM_CLASSES, B), jnp.int32),
        compiler_params=pltpu.CompilerParams(allow_input_fusion=[True]),
    )(xt)
    return jnp.transpose(out_t, (2, 0, 1))


# speedup vs baseline: 1.9794x; 1.9794x over previous
import jax
import jax.numpy as jnp
from jax.experimental import pallas as pl
from jax.experimental.pallas import tpu as pltpu

NUM_CLASSES = 1000
B = 1024
S = 50


SBLK = 2


def _onehot_block(x_ref, o_ref):
    c = jax.lax.broadcasted_iota(jnp.int32, (SBLK, NUM_CLASSES, B), 1)
    o_ref[...] = (c == x_ref[...]).astype(jnp.int32)


def kernel(x):
    xt = x.T.reshape(S, 1, B)
    out_t = pl.pallas_call(
        _onehot_block,
        grid=(S // SBLK,),
        in_specs=[pl.BlockSpec((SBLK, 1, B), lambda s: (s, 0, 0))],
        out_specs=pl.BlockSpec((SBLK, NUM_CLASSES, B), lambda s: (s, 0, 0)),
        out_shape=jax.ShapeDtypeStruct((S, NUM_CLASSES, B), jnp.int32),
        compiler_params=pltpu.CompilerParams(allow_input_fusion=[True]),
    )(xt)
    return jnp.transpose(out_t, (2, 0, 1))


# speedup vs baseline: 1.9964x; 1.0086x over previous
import jax
import jax.numpy as jnp
from jax.experimental import pallas as pl
from jax.experimental.pallas import tpu as pltpu

NUM_CLASSES = 1000
B = 1024
S = 50


def _onehot_block(x_ref, o_ref):
    c = jax.lax.broadcasted_iota(jnp.int32, (1, NUM_CLASSES, B), 1)
    o_ref[...] = (c == x_ref[...]).astype(jnp.int32)


def kernel(x):
    xt = x.T.reshape(S, 1, B)
    out_t = pl.pallas_call(
        _onehot_block,
        grid=(S,),
        in_specs=[pl.BlockSpec((1, 1, B), lambda s: (s, 0, 0))],
        out_specs=pl.BlockSpec((1, NUM_CLASSES, B), lambda s: (s, 0, 0)),
        out_shape=jax.ShapeDtypeStruct((S, NUM_CLASSES, B), jnp.int32),
        compiler_params=pltpu.CompilerParams(allow_input_fusion=[True]),
    )(xt)
    return jnp.transpose(out_t, (2, 0, 1))
